# baseline (speedup 1.0000x reference)
import functools

import jax
import jax.numpy as jnp
from jax import lax
from jax.experimental import pallas as pl
from jax.experimental.pallas import tpu as pltpu
from jax.experimental.pallas import tpu_sc as plsc

B, N, D = 2, 2048, 1024
E = 8
F = 4096
C = 320
BN = B * N
S = E * C
SP = S + 8
SZ = S + C
NCHUNK = 512
FC = 2048

_INTERPRET_TC = False


def _router_body(x_ref, wr_ref, sidx_ref, gidx_ref, w_ref, counts_ref,
                 aux_ref, z_ref):
    x = x_ref[...]
    wr = wr_ref[...]
    logits = jnp.dot(x.astype(jnp.bfloat16), wr.astype(jnp.bfloat16),
                     preferred_element_type=jnp.float32)

    m = jnp.max(logits, axis=-1, keepdims=True)
    p = jnp.exp(logits - m)
    s = jnp.sum(p, axis=-1, keepdims=True)
    z = m + jnp.log(s)
    gate = 1.0 / s
    probs = p / s

    ids = lax.broadcasted_iota(jnp.int32, (BN, E), 1)
    idx = jnp.min(jnp.where(logits == m, ids, E), axis=-1,
                  keepdims=True)
    onehot = (ids == idx).astype(jnp.float32)

    rows = lax.broadcasted_iota(jnp.int32, (BN, E), 0)
    b0 = (rows < N).astype(jnp.float32)
    dens0 = jnp.sum(onehot * b0, axis=0, keepdims=True) / N
    dens1 = jnp.sum(onehot * (1.0 - b0), axis=0, keepdims=True) / N
    prox0 = jnp.sum(probs * b0, axis=0, keepdims=True) / N
    prox1 = jnp.sum(probs * (1.0 - b0), axis=0, keepdims=True) / N
    aux = E * 0.5 * (jnp.sum(dens0 * prox0) + jnp.sum(dens1 * prox1))
    aux_ref[...] = aux.reshape(1, 1)
    z_ref[...] = (jnp.sum(z * z) / BN).reshape(1, 1)

    tri = (lax.broadcasted_iota(jnp.int32, (NCHUNK, NCHUNK), 0)
           >= lax.broadcasted_iota(jnp.int32, (NCHUNK, NCHUNK), 1)
           ).astype(jnp.bfloat16)
    carry = jnp.zeros((1, E), jnp.float32)
    cs_chunks = []
    for c in range(BN // NCHUNK):
        if c == N // NCHUNK:
            carry = jnp.zeros((1, E), jnp.float32)
        ohc = onehot[c * NCHUNK:(c + 1) * NCHUNK, :]
        csc = jnp.dot(tri, ohc.astype(jnp.bfloat16),
                      preferred_element_type=jnp.float32) + carry
        carry = csc[NCHUNK - 1:NCHUNK, :]
        cs_chunks.append(csc)
    cs = jnp.concatenate(cs_chunks, axis=0)

    pos = jnp.sum(onehot * cs, axis=-1, keepdims=True)
    keep = pos <= C
    w_ref[...] = jnp.where(keep, gate, 0.0)

    cnt0 = jnp.minimum(dens0 * N, float(C))
    cnt1 = jnp.minimum(dens1 * N, float(C))
    counts_ref[...] = jnp.concatenate([cnt0, cnt1], axis=0).astype(jnp.int32)

    dst = idx * C + (pos.astype(jnp.int32) - 1)
    dstk = jnp.where(keep, dst, S)
    bnum = lax.broadcasted_iota(jnp.int32, (BN, 1), 0) >= N
    sidx_ref[...] = dstk + jnp.where(bnum, SP, 0)
    gidx_ref[...] = dstk + jnp.where(bnum, SZ, 0)


def _router(x2d, Wr):
    return pl.pallas_call(
        _router_body,
        out_shape=(
            jax.ShapeDtypeStruct((BN, 1), jnp.int32),
            jax.ShapeDtypeStruct((BN, 1), jnp.int32),
            jax.ShapeDtypeStruct((BN, 1), jnp.float32),
            jax.ShapeDtypeStruct((B, E), jnp.int32),
            jax.ShapeDtypeStruct((1, 1), jnp.float32),
            jax.ShapeDtypeStruct((1, 1), jnp.float32),
        ),
        interpret=_INTERPRET_TC,
    )(x2d, Wr)


def _dispatch(x2d, sidx, w):
    info = plsc.get_sparse_core_info()
    nw = info.num_cores * info.num_subcores
    tpw = BN // nw
    nj = 4
    rpj = tpw // nj
    mesh = plsc.VectorSubcoreMesh(core_axis_name="c", subcore_axis_name="s")

    @functools.partial(
        pl.kernel, mesh=mesh,
        out_type=(
            jax.ShapeDtypeStruct((B * SP, D), jnp.float32),
            jax.ShapeDtypeStruct((B * SP,), jnp.float32),
        ),
        scratch_types=[
            pltpu.VMEM((nj, rpj), jnp.int32),
            pltpu.VMEM((nj, rpj), jnp.float32),
            pltpu.VMEM((rpj, D), jnp.float32),
            pltpu.SemaphoreType.DMA,
            pltpu.SemaphoreType.DMA,
        ],
    )
    def disp(x_hbm, sidx_hbm, w_hbm, ei_hbm, ws_hbm, idx_v, wv, buf_v,
             sem, semw):
        wid = lax.axis_index("s") * info.num_cores + lax.axis_index("c")
        pltpu.sync_copy(sidx_hbm.at[wid], idx_v)
        pltpu.sync_copy(w_hbm.at[wid], wv)
        base = wid * tpw
        for j in range(nj):
            cw = pltpu.async_copy(wv.at[j], ws_hbm.at[idx_v.at[j]], semw)
            pltpu.sync_copy(x_hbm.at[pl.ds(base + j * rpj, rpj)], buf_v)
            pltpu.async_copy(buf_v, ei_hbm.at[idx_v.at[j]], sem).wait()
            cw.wait()

    return disp(x2d, sidx.reshape(nw, nj, rpj), w.reshape(nw, nj, rpj))


def _ffn_body(counts_ref, ei_ref, ws_ref, w1_ref, w2_ref, eo_ref, acc_ref):
    e = pl.program_id(0)
    f = pl.program_id(1)
    nf = pl.num_programs(1)

    @pl.when(e < E)
    def _compute():
        x3 = ei_ref[...]
        riota = lax.broadcasted_iota(jnp.int32, (C, 1), 0)
        x0 = jnp.where(riota < counts_ref[0, e], x3[0], 0.0)
        x1 = jnp.where(riota < counts_ref[1, e], x3[1], 0.0)
        xb = jnp.concatenate([x0, x1], axis=0).astype(jnp.bfloat16)
        h = jnp.maximum(
            jnp.dot(xb, w1_ref[0].astype(jnp.bfloat16),
                    preferred_element_type=jnp.float32), 0.0)
        acc = jnp.dot(h.astype(jnp.bfloat16), w2_ref[0].astype(jnp.bfloat16),
                      preferred_element_type=jnp.float32)
        acc3 = acc.reshape(B, C, D)

        @pl.when(f == 0)
        def _():
            acc_ref[...] = acc3

        @pl.when(f > 0)
        def _():
            acc_ref[...] += acc3

        @pl.when(f == nf - 1)
        def _():
            ws = ws_ref[...]
            ws0 = jnp.where(riota < counts_ref[0, e], ws[0], 0.0)
            ws1 = jnp.where(riota < counts_ref[1, e], ws[1], 0.0)
            wsb = jnp.stack([ws0, ws1]).astype(jnp.bfloat16).astype(jnp.float32)
            ab = acc_ref[...].astype(jnp.bfloat16).astype(jnp.float32)
            eo_ref[...] = ab * wsb

    @pl.when(jnp.logical_and(e == E, f == 0))
    def _zeros():
        eo_ref[...] = jnp.zeros((B, C, D), jnp.float32)


def _ffn(ei, ws, counts, W1, W2):
    nf = F // FC
    return pl.pallas_call(
        _ffn_body,
        grid=(E + 1, nf),
        in_specs=[
            pl.BlockSpec(memory_space=pltpu.SMEM),
            pl.BlockSpec((B, C, D), lambda e, f: (0, jnp.minimum(e, E - 1), 0)),
            pl.BlockSpec((B, C, 1), lambda e, f: (0, jnp.minimum(e, E - 1), 0)),
            pl.BlockSpec((1, D, FC), lambda e, f: (jnp.minimum(e, E - 1), 0, f)),
            pl.BlockSpec((1, FC, D), lambda e, f: (jnp.minimum(e, E - 1), f, 0)),
        ],
        out_specs=pl.BlockSpec((B, C, D), lambda e, f: (0, e, 0)),
        out_shape=jax.ShapeDtypeStruct((B, SZ // C * C, D), jnp.float32),
        scratch_shapes=[pltpu.VMEM((B, C, D), jnp.float32)],
        interpret=_INTERPRET_TC,
    )(counts, ei.reshape(B, SP, D), ws.reshape(B, SP, 1), W1, W2)


def _combine(eo2d, gidx):
    info = plsc.get_sparse_core_info()
    nw = info.num_cores * info.num_subcores
    tpw = BN // nw
    nj = 4
    rpj = tpw // nj
    mesh = plsc.VectorSubcoreMesh(core_axis_name="c", subcore_axis_name="s")

    @functools.partial(
        pl.kernel, mesh=mesh,
        out_type=jax.ShapeDtypeStruct((BN, D), jnp.float32),
        scratch_types=[
            pltpu.VMEM((nj, rpj), jnp.int32),
            pltpu.VMEM((rpj, D), jnp.float32),
            pltpu.SemaphoreType.DMA,
        ],
    )
    def comb(eo_hbm, gidx_hbm, out_hbm, idx_v, buf_v, sem):
        wid = lax.axis_index("s") * info.num_cores + lax.axis_index("c")
        pltpu.sync_copy(gidx_hbm.at[wid], idx_v)
        base = wid * tpw
        for j in range(nj):
            pltpu.async_copy(eo_hbm.at[idx_v.at[j]], buf_v, sem).wait()
            pltpu.sync_copy(buf_v, out_hbm.at[pl.ds(base + j * rpj, rpj)])

    return comb(eo2d, gidx.reshape(nw, nj, rpj))


def kernel(token_inputs, Wr, W1, W2):
    x2d = token_inputs.reshape(BN, D)
    sidx, gidx, w, counts, aux, z = _router(x2d, Wr)
    ei, ws = _dispatch(x2d, sidx.reshape(BN), w.reshape(BN))
    eo = _ffn(ei, ws, counts, W1, W2)
    out = _combine(eo.reshape(B * SZ, D), gidx.reshape(BN))
    return (out.reshape(B, N, D), aux[0, 0], z[0, 0])


# speedup vs baseline: 1.1113x; 1.1113x over previous
import functools

import jax
import jax.numpy as jnp
from jax import lax
from jax.experimental import pallas as pl
from jax.experimental.pallas import tpu as pltpu
from jax.experimental.pallas import tpu_sc as plsc

B, N, D = 2, 2048, 1024
E = 8
F = 4096
C = 320
BN = B * N
S = E * C
SP = S + 8
SZ = S + C
NCHUNK = 512
FC = 2048

_INTERPRET_TC = False


def _router_body(x_ref, wr_ref, sidx_ref, gidx_ref, w_ref, xs_ref, counts_ref,
                 aux_ref, z_ref):
    x = x_ref[...]
    wr = wr_ref[...]
    logits = jnp.dot(x.astype(jnp.bfloat16), wr.astype(jnp.bfloat16),
                     preferred_element_type=jnp.float32)

    m = jnp.max(logits, axis=-1, keepdims=True)
    p = jnp.exp(logits - m)
    s = jnp.sum(p, axis=-1, keepdims=True)
    z = m + jnp.log(s)
    gate = 1.0 / s
    probs = p / s

    ids = lax.broadcasted_iota(jnp.int32, (BN, E), 1)
    idx = jnp.min(jnp.where(logits == m, ids, E), axis=-1,
                  keepdims=True)
    onehot = (ids == idx).astype(jnp.float32)

    rows = lax.broadcasted_iota(jnp.int32, (BN, E), 0)
    b0 = (rows < N).astype(jnp.float32)
    dens0 = jnp.sum(onehot * b0, axis=0, keepdims=True) / N
    dens1 = jnp.sum(onehot * (1.0 - b0), axis=0, keepdims=True) / N
    prox0 = jnp.sum(probs * b0, axis=0, keepdims=True) / N
    prox1 = jnp.sum(probs * (1.0 - b0), axis=0, keepdims=True) / N
    aux = E * 0.5 * (jnp.sum(dens0 * prox0) + jnp.sum(dens1 * prox1))
    aux_ref[...] = aux.reshape(1, 1)
    z_ref[...] = (jnp.sum(z * z) / BN).reshape(1, 1)

    tri = (lax.broadcasted_iota(jnp.int32, (NCHUNK, NCHUNK), 0)
           >= lax.broadcasted_iota(jnp.int32, (NCHUNK, NCHUNK), 1)
           ).astype(jnp.bfloat16)
    carry = jnp.zeros((1, E), jnp.float32)
    cs_chunks = []
    for c in range(BN // NCHUNK):
        if c == N // NCHUNK:
            carry = jnp.zeros((1, E), jnp.float32)
        ohc = onehot[c * NCHUNK:(c + 1) * NCHUNK, :]
        csc = jnp.dot(tri, ohc.astype(jnp.bfloat16),
                      preferred_element_type=jnp.float32) + carry
        carry = csc[NCHUNK - 1:NCHUNK, :]
        cs_chunks.append(csc)
    cs = jnp.concatenate(cs_chunks, axis=0)

    pos = jnp.sum(onehot * cs, axis=-1, keepdims=True)
    keep = pos <= C
    w = jnp.where(keep, gate, 0.0)
    w_ref[...] = w
    xs_ref[...] = x * w

    cnt0 = jnp.minimum(dens0 * N, float(C))
    cnt1 = jnp.minimum(dens1 * N, float(C))
    counts_ref[...] = jnp.concatenate([cnt0, cnt1], axis=0).astype(jnp.int32)

    dst = idx * C + (pos.astype(jnp.int32) - 1)
    dstk = jnp.where(keep, dst, S)
    bnum = lax.broadcasted_iota(jnp.int32, (BN, 1), 0) >= N
    sidx_ref[...] = dstk + jnp.where(bnum, SP, 0)
    gidx_ref[...] = dstk + jnp.where(bnum, SZ, 0)


def _router(x2d, Wr):
    return pl.pallas_call(
        _router_body,
        out_shape=(
            jax.ShapeDtypeStruct((BN, 1), jnp.int32),
            jax.ShapeDtypeStruct((BN, 1), jnp.int32),
            jax.ShapeDtypeStruct((BN, 1), jnp.float32),
            jax.ShapeDtypeStruct((BN, D), jnp.float32),
            jax.ShapeDtypeStruct((B, E), jnp.int32),
            jax.ShapeDtypeStruct((1, 1), jnp.float32),
            jax.ShapeDtypeStruct((1, 1), jnp.float32),
        ),
        interpret=_INTERPRET_TC,
    )(x2d, Wr)


def _dispatch(x2d, sidx):
    info = plsc.get_sparse_core_info()
    nw = info.num_cores * info.num_subcores
    tpw = BN // nw
    nj = 4
    rpj = tpw // nj
    mesh = plsc.VectorSubcoreMesh(core_axis_name="c", subcore_axis_name="s")

    @functools.partial(
        pl.kernel, mesh=mesh,
        out_type=jax.ShapeDtypeStruct((B * SP, D), jnp.float32),
        scratch_types=[
            pltpu.VMEM((nj, rpj), jnp.int32),
            pltpu.VMEM((rpj, D), jnp.float32),
            pltpu.SemaphoreType.DMA,
        ],
    )
    def disp(x_hbm, sidx_hbm, ei_hbm, idx_v, buf_v, sem):
        wid = lax.axis_index("s") * info.num_cores + lax.axis_index("c")
        pltpu.sync_copy(sidx_hbm.at[wid], idx_v)
        base = wid * tpw
        for j in range(nj):
            pltpu.sync_copy(x_hbm.at[pl.ds(base + j * rpj, rpj)], buf_v)
            pltpu.async_copy(buf_v, ei_hbm.at[idx_v.at[j]], sem).wait()

    return disp(x2d, sidx.reshape(nw, nj, rpj))


def _ffn_body(counts_ref, ei_ref, w1_ref, w2_ref, eo_ref):
    e = pl.program_id(0)
    f = pl.program_id(1)

    @pl.when(e < E)
    def _compute():
        x3 = ei_ref[...]
        riota = lax.broadcasted_iota(jnp.int32, (C, 1), 0)
        x0 = jnp.where(riota < counts_ref[0, e], x3[0], 0.0)
        x1 = jnp.where(riota < counts_ref[1, e], x3[1], 0.0)
        xb = jnp.concatenate([x0, x1], axis=0).astype(jnp.bfloat16)
        h = jnp.maximum(
            jnp.dot(xb, w1_ref[0].astype(jnp.bfloat16),
                    preferred_element_type=jnp.float32), 0.0)
        acc = jnp.dot(h.astype(jnp.bfloat16), w2_ref[0].astype(jnp.bfloat16),
                      preferred_element_type=jnp.float32)
        acc3 = acc.reshape(B, C, D)

        @pl.when(f == 0)
        def _():
            eo_ref[...] = acc3

        @pl.when(f > 0)
        def _():
            eo_ref[...] += acc3

    @pl.when(jnp.logical_and(e == E, f == 0))
    def _zeros():
        eo_ref[...] = jnp.zeros((B, C, D), jnp.float32)


def _ffn(ei, counts, W1, W2):
    nf = F // FC
    return pl.pallas_call(
        _ffn_body,
        grid=(E + 1, nf),
        in_specs=[
            pl.BlockSpec(memory_space=pltpu.SMEM),
            pl.BlockSpec((B, C, D), lambda e, f: (0, jnp.minimum(e, E - 1), 0)),
            pl.BlockSpec((1, D, FC), lambda e, f: (jnp.minimum(e, E - 1), 0, f)),
            pl.BlockSpec((1, FC, D), lambda e, f: (jnp.minimum(e, E - 1), f, 0)),
        ],
        out_specs=pl.BlockSpec((B, C, D), lambda e, f: (0, e, 0)),
        out_shape=jax.ShapeDtypeStruct((B, SZ // C * C, D), jnp.float32),
        interpret=_INTERPRET_TC,
    )(counts, ei.reshape(B, SP, D), W1, W2)


def _combine(eo2d, gidx):
    info = plsc.get_sparse_core_info()
    nw = info.num_cores * info.num_subcores
    tpw = BN // nw
    nj = 4
    rpj = tpw // nj
    mesh = plsc.VectorSubcoreMesh(core_axis_name="c", subcore_axis_name="s")

    @functools.partial(
        pl.kernel, mesh=mesh,
        out_type=jax.ShapeDtypeStruct((BN, D), jnp.float32),
        scratch_types=[
            pltpu.VMEM((nj, rpj), jnp.int32),
            pltpu.VMEM((rpj, D), jnp.float32),
            pltpu.SemaphoreType.DMA,
        ],
    )
    def comb(eo_hbm, gidx_hbm, out_hbm, idx_v, buf_v, sem):
        wid = lax.axis_index("s") * info.num_cores + lax.axis_index("c")
        pltpu.sync_copy(gidx_hbm.at[wid], idx_v)
        base = wid * tpw
        for j in range(nj):
            pltpu.async_copy(eo_hbm.at[idx_v.at[j]], buf_v, sem).wait()
            pltpu.sync_copy(buf_v, out_hbm.at[pl.ds(base + j * rpj, rpj)])

    return comb(eo2d, gidx.reshape(nw, nj, rpj))


def kernel(token_inputs, Wr, W1, W2):
    x2d = token_inputs.reshape(BN, D)
    sidx, gidx, w, xs, counts, aux, z = _router(x2d, Wr)
    ei = _dispatch(xs, sidx.reshape(BN))
    eo = _ffn(ei, counts, W1, W2)
    out = _combine(eo.reshape(B * SZ, D), gidx.reshape(BN))
    return (out.reshape(B, N, D), aux[0, 0], z[0, 0])


# speedup vs baseline: 1.1171x; 1.0052x over previous
import functools

import jax
import jax.numpy as jnp
from jax import lax
from jax.experimental import pallas as pl
from jax.experimental.pallas import tpu as pltpu
from jax.experimental.pallas import tpu_sc as plsc

B, N, D = 2, 2048, 1024
E = 8
F = 4096
C = 320
BN = B * N
S = E * C
SP = S + 8
SZ = S + C
NCHUNK = 512
FC = 2048

_INTERPRET_TC = False


def _router_body(x_ref, wr_ref, sidx_ref, gidx_ref, w_ref, xs_ref, counts_ref,
                 aux_ref, z_ref):
    x = x_ref[...]
    wr = wr_ref[...]
    logits = jnp.dot(x.astype(jnp.bfloat16), wr.astype(jnp.bfloat16),
                     preferred_element_type=jnp.float32)

    m = jnp.max(logits, axis=-1, keepdims=True)
    p = jnp.exp(logits - m)
    s = jnp.sum(p, axis=-1, keepdims=True)
    z = m + jnp.log(s)
    gate = 1.0 / s
    probs = p / s

    ids = lax.broadcasted_iota(jnp.int32, (BN, E), 1)
    idx = jnp.min(jnp.where(logits == m, ids, E), axis=-1,
                  keepdims=True)
    onehot = (ids == idx).astype(jnp.float32)

    rows = lax.broadcasted_iota(jnp.int32, (BN, E), 0)
    b0 = (rows < N).astype(jnp.float32)
    dens0 = jnp.sum(onehot * b0, axis=0, keepdims=True) / N
    dens1 = jnp.sum(onehot * (1.0 - b0), axis=0, keepdims=True) / N
    prox0 = jnp.sum(probs * b0, axis=0, keepdims=True) / N
    prox1 = jnp.sum(probs * (1.0 - b0), axis=0, keepdims=True) / N
    aux = E * 0.5 * (jnp.sum(dens0 * prox0) + jnp.sum(dens1 * prox1))
    aux_ref[...] = aux.reshape(1, 1)
    z_ref[...] = (jnp.sum(z * z) / BN).reshape(1, 1)

    tri = (lax.broadcasted_iota(jnp.int32, (NCHUNK, NCHUNK), 0)
           >= lax.broadcasted_iota(jnp.int32, (NCHUNK, NCHUNK), 1)
           ).astype(jnp.bfloat16)
    carry = jnp.zeros((1, E), jnp.float32)
    cs_chunks = []
    for c in range(BN // NCHUNK):
        if c == N // NCHUNK:
            carry = jnp.zeros((1, E), jnp.float32)
        ohc = onehot[c * NCHUNK:(c + 1) * NCHUNK, :]
        csc = jnp.dot(tri, ohc.astype(jnp.bfloat16),
                      preferred_element_type=jnp.float32) + carry
        carry = csc[NCHUNK - 1:NCHUNK, :]
        cs_chunks.append(csc)
    cs = jnp.concatenate(cs_chunks, axis=0)

    pos = jnp.sum(onehot * cs, axis=-1, keepdims=True)
    keep = pos <= C
    w = jnp.where(keep, gate, 0.0)
    w_ref[...] = w
    xs_ref[...] = x * w

    cnt0 = jnp.minimum(dens0 * N, float(C))
    cnt1 = jnp.minimum(dens1 * N, float(C))
    counts_ref[...] = jnp.concatenate([cnt0, cnt1], axis=0).astype(jnp.int32)

    dst = idx * C + (pos.astype(jnp.int32) - 1)
    dstk = jnp.where(keep, dst, S)
    bnum = lax.broadcasted_iota(jnp.int32, (BN, 1), 0) >= N
    sidx_ref[...] = dstk + jnp.where(bnum, SP, 0)
    gidx_ref[...] = dstk + jnp.where(bnum, SZ, 0)


def _router(x2d, Wr):
    return pl.pallas_call(
        _router_body,
        out_shape=(
            jax.ShapeDtypeStruct((BN, 1), jnp.int32),
            jax.ShapeDtypeStruct((BN, 1), jnp.int32),
            jax.ShapeDtypeStruct((BN, 1), jnp.float32),
            jax.ShapeDtypeStruct((BN, D), jnp.float32),
            jax.ShapeDtypeStruct((B, E), jnp.int32),
            jax.ShapeDtypeStruct((1, 1), jnp.float32),
            jax.ShapeDtypeStruct((1, 1), jnp.float32),
        ),
        interpret=_INTERPRET_TC,
    )(x2d, Wr)


def _dispatch(x2d, sidx):
    info = plsc.get_sparse_core_info()
    nw = info.num_cores * info.num_subcores
    tpw = BN // nw
    nj = 4
    rpj = tpw // nj
    mesh = plsc.VectorSubcoreMesh(core_axis_name="c", subcore_axis_name="s")

    @functools.partial(
        pl.kernel, mesh=mesh,
        out_type=jax.ShapeDtypeStruct((B * SP, D), jnp.float32),
        scratch_types=[
            pltpu.VMEM((nj, rpj), jnp.int32),
            pltpu.VMEM((rpj, D), jnp.float32),
            pltpu.VMEM((rpj, D), jnp.float32),
            pltpu.SemaphoreType.DMA,
            pltpu.SemaphoreType.DMA,
        ],
    )
    def disp(x_hbm, sidx_hbm, ei_hbm, idx_v, buf0, buf1, sem_l, sem_s):
        wid = lax.axis_index("s") * info.num_cores + lax.axis_index("c")
        pltpu.sync_copy(sidx_hbm.at[wid], idx_v)
        base = wid * tpw
        bufs = (buf0, buf1)
        loads = [None] * nj
        stores = [None] * nj
        loads[0] = pltpu.async_copy(x_hbm.at[pl.ds(base, rpj)], bufs[0], sem_l)
        for j in range(nj):
            loads[j].wait()
            stores[j] = pltpu.async_copy(bufs[j % 2], ei_hbm.at[idx_v.at[j]],
                                         sem_s)
            if j + 1 < nj:
                if j >= 1:
                    stores[j - 1].wait()
                loads[j + 1] = pltpu.async_copy(
                    x_hbm.at[pl.ds(base + (j + 1) * rpj, rpj)],
                    bufs[(j + 1) % 2], sem_l)
        stores[nj - 2].wait()
        stores[nj - 1].wait()

    return disp(x2d, sidx.reshape(nw, nj, rpj))


def _ffn_body(counts_ref, ei_ref, w1_ref, w2_ref, eo_ref):
    e = pl.program_id(0)
    f = pl.program_id(1)

    @pl.when(e < E)
    def _compute():
        x3 = ei_ref[...]
        riota = lax.broadcasted_iota(jnp.int32, (C, 1), 0)
        x0 = jnp.where(riota < counts_ref[0, e], x3[0], 0.0)
        x1 = jnp.where(riota < counts_ref[1, e], x3[1], 0.0)
        xb = jnp.concatenate([x0, x1], axis=0).astype(jnp.bfloat16)
        h = jnp.maximum(
            jnp.dot(xb, w1_ref[0].astype(jnp.bfloat16),
                    preferred_element_type=jnp.float32), 0.0)
        acc = jnp.dot(h.astype(jnp.bfloat16), w2_ref[0].astype(jnp.bfloat16),
                      preferred_element_type=jnp.float32)
        acc3 = acc.reshape(B, C, D)

        @pl.when(f == 0)
        def _():
            eo_ref[...] = acc3

        @pl.when(f > 0)
        def _():
            eo_ref[...] += acc3

    @pl.when(jnp.logical_and(e == E, f == 0))
    def _zeros():
        eo_ref[...] = jnp.zeros((B, C, D), jnp.float32)


def _ffn(ei, counts, W1, W2):
    nf = F // FC
    return pl.pallas_call(
        _ffn_body,
        grid=(E + 1, nf),
        in_specs=[
            pl.BlockSpec(memory_space=pltpu.SMEM),
            pl.BlockSpec((B, C, D), lambda e, f: (0, jnp.minimum(e, E - 1), 0)),
            pl.BlockSpec((1, D, FC), lambda e, f: (jnp.minimum(e, E - 1), 0, f)),
            pl.BlockSpec((1, FC, D), lambda e, f: (jnp.minimum(e, E - 1), f, 0)),
        ],
        out_specs=pl.BlockSpec((B, C, D), lambda e, f: (0, e, 0)),
        out_shape=jax.ShapeDtypeStruct((B, SZ // C * C, D), jnp.float32),
        interpret=_INTERPRET_TC,
    )(counts, ei.reshape(B, SP, D), W1, W2)


def _combine(eo2d, gidx):
    info = plsc.get_sparse_core_info()
    nw = info.num_cores * info.num_subcores
    tpw = BN // nw
    nj = 4
    rpj = tpw // nj
    mesh = plsc.VectorSubcoreMesh(core_axis_name="c", subcore_axis_name="s")

    @functools.partial(
        pl.kernel, mesh=mesh,
        out_type=jax.ShapeDtypeStruct((BN, D), jnp.float32),
        scratch_types=[
            pltpu.VMEM((nj, rpj), jnp.int32),
            pltpu.VMEM((rpj, D), jnp.float32),
            pltpu.VMEM((rpj, D), jnp.float32),
            pltpu.SemaphoreType.DMA,
            pltpu.SemaphoreType.DMA,
        ],
    )
    def comb(eo_hbm, gidx_hbm, out_hbm, idx_v, buf0, buf1, sem_g, sem_w):
        wid = lax.axis_index("s") * info.num_cores + lax.axis_index("c")
        pltpu.sync_copy(gidx_hbm.at[wid], idx_v)
        base = wid * tpw
        bufs = (buf0, buf1)
        gathers = [None] * nj
        writes = [None] * nj
        gathers[0] = pltpu.async_copy(eo_hbm.at[idx_v.at[0]], bufs[0], sem_g)
        for j in range(nj):
            gathers[j].wait()
            writes[j] = pltpu.async_copy(
                bufs[j % 2], out_hbm.at[pl.ds(base + j * rpj, rpj)], sem_w)
            if j + 1 < nj:
                if j >= 1:
                    writes[j - 1].wait()
                gathers[j + 1] = pltpu.async_copy(eo_hbm.at[idx_v.at[j + 1]],
                                                  bufs[(j + 1) % 2], sem_g)
        writes[nj - 2].wait()
        writes[nj - 1].wait()

    return comb(eo2d, gidx.reshape(nw, nj, rpj))


def kernel(token_inputs, Wr, W1, W2):
    x2d = token_inputs.reshape(BN, D)
    sidx, gidx, w, xs, counts, aux, z = _router(x2d, Wr)
    ei = _dispatch(xs, sidx.reshape(BN))
    eo = _ffn(ei, counts, W1, W2)
    out = _combine(eo.reshape(B * SZ, D), gidx.reshape(BN))
    return (out.reshape(B, N, D), aux[0, 0], z[0, 0])
